# Initial kernel scaffold; baseline (speedup 1.0000x reference)
import jax
import jax.numpy as jnp
from jax.experimental import pallas as pl


def kernel(node_s, node_v, edge_s, edge_v, edge_index, seq, params):
    raise NotImplementedError("write your pallas kernel here")


# speedup vs baseline: 8.5223x; 8.5223x over previous
import functools
import jax
import jax.numpy as jnp
from jax import lax
from jax.experimental import pallas as pl
from jax.experimental.pallas import tpu as pltpu
from jax.experimental.pallas import tpu_sc as plsc

NN = 10000
NE = 160000
DS = 100
KV = 16
C_V = 100
C_ONE = 148
DTOK = 20
D_ENC = 160
D_DEC = 176
BN = 1000
BE = 2000
EPS = 1e-8


def _gvp_tc(v3, s_in, whT, wsT, b, wvT, relu_s, gate_v):
    vh = [x @ whT for x in v3]
    vn = jnp.sqrt(jnp.clip(vh[0] * vh[0] + vh[1] * vh[1] + vh[2] * vh[2], EPS))
    s = jnp.concatenate([s_in, vn], axis=1) @ wsT + b
    if relu_s:
        s = jnp.maximum(s, 0.0)
    vo = None
    if wvT is not None:
        vo = [h @ wvT for h in vh]
        if gate_v:
            g = jax.nn.sigmoid(jnp.sqrt(jnp.clip(
                vo[0] * vo[0] + vo[1] * vo[1] + vo[2] * vo[2], EPS)))
            vo = [x * g for x in vo]
    return s, vo


def _ln_tc(s, v3, gamma, beta):
    mu = jnp.mean(s, axis=1, keepdims=True)
    var = jnp.mean((s - mu) * (s - mu), axis=1, keepdims=True)
    s = (s - mu) * lax.rsqrt(var + 1e-5) * gamma + beta
    nsq = jnp.clip(v3[0] * v3[0] + v3[1] * v3[1] + v3[2] * v3[2], EPS)
    vn = jnp.sqrt(jnp.mean(nsq, axis=1, keepdims=True))
    return s, [x / vn for x in v3]


def _vslices(x, col, k):
    return [x[:, col + k * c:col + k * (c + 1)] for c in range(3)]


def _full(shape):
    nd = len(shape)
    return pl.BlockSpec(shape, lambda i: (0,) * nd)


def _rows(block, width):
    return pl.BlockSpec((block, width), lambda i: (i, 0))


def _gvp_w(p):
    wvT = jnp.asarray(p['wv']).T if 'wv' in p else None
    return [jnp.asarray(p['wh']).T, jnp.asarray(p['ws_w']).T,
            jnp.asarray(p['ws_b'])[None, :], wvT]


def _embed_node(node_s, nv3, params):
    w = _gvp_w(params['W_v'])
    g = params['ln_v']['gamma'][None, :]
    bt = params['ln_v']['beta'][None, :]

    def body(ns_ref, nv_ref, whT, wsT, b, wvT, lng, lnb, out_ref):
        v3 = _vslices(nv_ref[...], 0, 3)
        s, v = _gvp_tc(v3, ns_ref[...], whT[...], wsT[...], b[...], wvT[...],
                       False, False)
        s, v = _ln_tc(s, v, lng[...], lnb[...])
        out_ref[:, :DS] = s
        for c in range(3):
            out_ref[:, C_V + KV * c:C_V + KV * (c + 1)] = v[c]
        out_ref[:, C_ONE:D_ENC] = jnp.zeros((s.shape[0], D_ENC - C_ONE), jnp.float32)

    args = [node_s, nv3] + w + [g, bt]
    return pl.pallas_call(
        body, grid=(NN // BN,),
        in_specs=[_rows(BN, 6), _rows(BN, 9)] + [_full(a.shape) for a in args[2:]],
        out_specs=_rows(BN, D_ENC),
        out_shape=jax.ShapeDtypeStruct((NN, D_ENC), jnp.float32),
    )(*args)


def _embed_edge(edge_s, ev3, params):
    w = _gvp_w(params['W_e'])
    g = params['ln_e']['gamma'][None, :]
    bt = params['ln_e']['beta'][None, :]

    def body(es_ref, ev_ref, whT, wsT, b, wvT, lng, lnb, so_ref, vo_ref):
        v3 = _vslices(ev_ref[...], 0, 1)
        s, v = _gvp_tc(v3, es_ref[...], whT[...], wsT[...], b[...], wvT[...],
                       False, False)
        s, v = _ln_tc(s, v, lng[...], lnb[...])
        so_ref[...] = s
        vo_ref[...] = jnp.concatenate(v, axis=1)

    args = [edge_s, ev3] + w + [g, bt]
    return pl.pallas_call(
        body, grid=(NE // BE,),
        in_specs=[_rows(BE, 32), _rows(BE, 3)] + [_full(a.shape) for a in args[2:]],
        out_specs=(_rows(BE, 32), _rows(BE, 3)),
        out_shape=(jax.ShapeDtypeStruct((NE, 32), jnp.float32),
                   jax.ShapeDtypeStruct((NE, 3), jnp.float32)),
    )(*args)


def _token_embed(seq, W_s):
    def body(seq_ref, ws_ref, out_ref):
        oh = (lax.broadcasted_iota(jnp.int32, (BN, 33), 1) == seq_ref[...])
        out_ref[...] = oh.astype(jnp.float32) @ ws_ref[...]

    return pl.pallas_call(
        body, grid=(NN // BN,),
        in_specs=[_rows(BN, 1), _full(W_s.shape)],
        out_specs=_rows(BN, DTOK),
        out_shape=jax.ShapeDtypeStruct((NN, DTOK), jnp.float32),
    )(seq.reshape(NN, 1), W_s)


def _dec_indices(src2, dst2):
    def body(s_ref, d_ref, oj_ref, oi_ref):
        s = s_ref[...]
        d = d_ref[...]
        fut = s < d
        off = jnp.where(fut, 0, NN)
        oj_ref[...] = s + off
        oi_ref[...] = d + off

    return pl.pallas_call(
        body, grid=(1,),
        in_specs=[_rows(1250, 128), _rows(1250, 128)],
        out_specs=(_rows(1250, 128), _rows(1250, 128)),
        out_shape=(jax.ShapeDtypeStruct((1250, 128), jnp.int32),
                   jax.ShapeDtypeStruct((1250, 128), jnp.int32)),
    )(src2, dst2)


def _message(dec, gj, gi, es, ev, mp):
    DT = D_DEC if dec else D_ENC
    w = _gvp_w(mp['g0']) + _gvp_w(mp['g1']) + _gvp_w(mp['g2'])

    def body(gj_ref, gi_ref, es_ref, ev_ref,
             wh0, ws0, b0, wv0, wh1, ws1, b1, wv1, wh2, ws2, b2, wv2,
             out_ref):
        gjv = gj_ref[...]
        giv = gi_ref[...]
        sj = gjv[:, :DS]
        si = giv[:, :DS]
        vj = _vslices(gjv, C_V, KV)
        vi = _vslices(giv, C_V, KV)
        ev_ = ev_ref[...]
        vcat = [jnp.concatenate([vj[c], ev_[:, c:c + 1], vi[c]], axis=1)
                for c in range(3)]
        if dec:
            s_in = jnp.concatenate(
                [sj, es_ref[...], gjv[:, C_ONE:C_ONE + DTOK], si], axis=1)
        else:
            s_in = jnp.concatenate([sj, es_ref[...], si], axis=1)
        s, v = _gvp_tc(vcat, s_in, wh0[...], ws0[...], b0[...], wv0[...], True, True)
        s, v = _gvp_tc(v, s, wh1[...], ws1[...], b1[...], wv1[...], True, True)
        s, v = _gvp_tc(v, s, wh2[...], ws2[...], b2[...], wv2[...], False, False)
        out_ref[:, :DS] = s
        for c in range(3):
            out_ref[:, C_V + KV * c:C_V + KV * (c + 1)] = v[c]
        B = s.shape[0]
        out_ref[:, C_ONE:C_ONE + 1] = jnp.ones((B, 1), jnp.float32)
        out_ref[:, C_ONE + 1:D_ENC] = jnp.zeros((B, D_ENC - C_ONE - 1), jnp.float32)

    args = [gj, gi, es, ev] + w
    return pl.pallas_call(
        body, grid=(NE // BE,),
        in_specs=[_rows(BE, DT), _rows(BE, DT), _rows(BE, 32), _rows(BE, 3)]
                 + [_full(a.shape) for a in w],
        out_specs=_rows(BE, D_ENC),
        out_shape=jax.ShapeDtypeStruct((NE, D_ENC), jnp.float32),
    )(*args)


def _node_update(Din, Dout, tok, tbl, a0, a1, htok, lp):
    w = (_gvp_w(lp['ff0']) + _gvp_w(lp['ff1'])
         + [lp['norm0']['gamma'][None, :], lp['norm0']['beta'][None, :],
            lp['norm1']['gamma'][None, :], lp['norm1']['beta'][None, :]])

    def body(*refs):
        if tok:
            (tbl_ref, a0_ref, a1_ref, tok_ref, whf0, wsf0, bf0, wvf0,
             whf1, wsf1, bf1, wvf1, g0, b0, g1, b1, out_ref) = refs
        else:
            (tbl_ref, a0_ref, a1_ref, whf0, wsf0, bf0, wvf0,
             whf1, wsf1, bf1, wvf1, g0, b0, g1, b1, out_ref) = refs
        tblv = tbl_ref[...]
        s = tblv[:, :DS]
        v = _vslices(tblv, C_V, KV)
        agg = a0_ref[...] + a1_ref[...]
        cnt = jnp.maximum(agg[:, C_ONE:C_ONE + 1], 1.0)
        s = s + agg[:, :DS] / cnt
        av = _vslices(agg, C_V, KV)
        v = [v[c] + av[c] / cnt for c in range(3)]
        s, v = _ln_tc(s, v, g0[...], b0[...])
        fs, fv = _gvp_tc(v, s, whf0[...], wsf0[...], bf0[...], wvf0[...], True, True)
        fs, fv = _gvp_tc(fv, fs, whf1[...], wsf1[...], bf1[...], wvf1[...],
                         False, False)
        s, v = _ln_tc(s + fs, [v[c] + fv[c] for c in range(3)], g1[...], b1[...])
        out_ref[:, :DS] = s
        for c in range(3):
            out_ref[:, C_V + KV * c:C_V + KV * (c + 1)] = v[c]
        if tok:
            out_ref[:, C_ONE:C_ONE + DTOK] = tok_ref[...]
            pad0 = C_ONE + DTOK
        else:
            pad0 = C_ONE
        out_ref[:, pad0:Dout] = jnp.zeros((s.shape[0], Dout - pad0), jnp.float32)

    args = [tbl, a0, a1] + ([htok] if tok else []) + w
    in_specs = ([_rows(BN, Din), _rows(BN, D_ENC), _rows(BN, D_ENC)]
                + ([_rows(BN, DTOK)] if tok else [])
                + [_full(a.shape) for a in w])
    return pl.pallas_call(
        body, grid=(NN // BN,),
        in_specs=in_specs,
        out_specs=_rows(BN, Dout),
        out_shape=jax.ShapeDtypeStruct((NN, Dout), jnp.float32),
    )(*args)


def _readout(tbl, params):
    whT = jnp.asarray(params['W_out']['wh']).T
    wsT = jnp.asarray(params['W_out']['ws_w']).T
    b = jnp.asarray(params['W_out']['ws_b'])[None, :]

    def body(tbl_ref, wh_ref, ws_ref, b_ref, lp_ref, lg_ref):
        tblv = tbl_ref[...]
        s = tblv[:, :DS]
        v = _vslices(tblv, C_V, KV)
        vh = [x @ wh_ref[...] for x in v]
        vn = jnp.sqrt(jnp.clip(vh[0] * vh[0] + vh[1] * vh[1] + vh[2] * vh[2], EPS))
        logits = jnp.concatenate([s, vn], axis=1) @ ws_ref[...] + b_ref[...]
        m = jnp.max(logits, axis=1, keepdims=True)
        lse = jnp.log(jnp.sum(jnp.exp(logits - m), axis=1, keepdims=True)) + m
        lg_ref[...] = logits
        lp_ref[...] = logits - lse

    return pl.pallas_call(
        body, grid=(NN // BN,),
        in_specs=[_rows(BN, D_ENC), _full(whT.shape), _full(wsT.shape),
                  _full(b.shape)],
        out_specs=(_rows(BN, 33), _rows(BN, 33)),
        out_shape=(jax.ShapeDtypeStruct((NN, 33), jnp.float32),
                   jax.ShapeDtypeStruct((NN, 33), jnp.float32)),
    )(tbl, whT, wsT, b)


_CH = 128
_NCHUNK = NE // _CH
_NW = 32
_PER = (_NCHUNK + _NW - 1) // _NW
_RSUB = NN // 16


def _make_gather(D):
    mesh = plsc.VectorSubcoreMesh(core_axis_name="c", subcore_axis_name="s")

    @functools.partial(
        pl.kernel,
        out_type=(jax.ShapeDtypeStruct((NE, D), jnp.float32),
                  jax.ShapeDtypeStruct((NE, D), jnp.float32)),
        mesh=mesh,
        compiler_params=pltpu.CompilerParams(use_tc_tiling_on_sc=False),
        scratch_types=[pltpu.VMEM((_CH,), jnp.int32),
                       pltpu.VMEM((_CH,), jnp.int32),
                       pltpu.VMEM((_CH, D), jnp.float32),
                       pltpu.VMEM((_CH, D), jnp.float32),
                       pltpu.SemaphoreType.DMA,
                       pltpu.SemaphoreType.DMA],
    )
    def k(tbl, idxj, idxi, gj, gi, bj, bi, rj, ri, smj, smi):
        wid = lax.axis_index("s") * 2 + lax.axis_index("c")

        def step(t, carry):
            ch = wid + t * _NW

            @pl.when(ch < _NCHUNK)
            def _():
                off = ch * _CH
                pltpu.sync_copy(idxj.at[pl.ds(off, _CH)], bj)
                pltpu.sync_copy(idxi.at[pl.ds(off, _CH)], bi)
                cj = pltpu.async_copy(tbl.at[bj], rj, smj)
                ci = pltpu.async_copy(tbl.at[bi], ri, smi)
                cj.wait()
                ci.wait()
                pltpu.sync_copy(rj, gj.at[pl.ds(off, _CH)])
                pltpu.sync_copy(ri, gi.at[pl.ds(off, _CH)])
            return carry

        lax.fori_loop(0, _PER, step, 0)

    return k


def _make_scatter():
    mesh = plsc.VectorSubcoreMesh(core_axis_name="c", subcore_axis_name="s")

    @functools.partial(
        pl.kernel,
        out_type=(jax.ShapeDtypeStruct((NN, D_ENC), jnp.float32),
                  jax.ShapeDtypeStruct((NN, D_ENC), jnp.float32)),
        mesh=mesh,
        compiler_params=pltpu.CompilerParams(use_tc_tiling_on_sc=False),
        scratch_types=[pltpu.VMEM((_CH,), jnp.int32),
                       pltpu.VMEM((_CH, D_ENC), jnp.float32),
                       pltpu.VMEM_SHARED((NN, D_ENC), jnp.float32)],
    )
    def k(msg, dst, zrows, out0, out1, bidx, bmsg, acc):
        cid = lax.axis_index("c")
        sid = lax.axis_index("s")
        wid = sid * 2 + cid
        row0 = sid * _RSUB
        pltpu.sync_copy(zrows, acc.at[pl.ds(row0, _RSUB)])
        plsc.subcore_barrier()

        def step(t, carry):
            ch = wid + t * _NW

            @pl.when(ch < _NCHUNK)
            def _():
                off = ch * _CH
                pltpu.sync_copy(dst.at[pl.ds(off, _CH)], bidx)
                pltpu.sync_copy(msg.at[pl.ds(off, _CH)], bmsg)
                pltpu.sync_copy(bmsg, acc.at[bidx], add=True)
            return carry

        lax.fori_loop(0, _PER, step, 0)
        plsc.subcore_barrier()

        @pl.when(cid == 0)
        def _():
            pltpu.sync_copy(acc.at[pl.ds(row0, _RSUB)], out0.at[pl.ds(row0, _RSUB)])

        @pl.when(cid == 1)
        def _():
            pltpu.sync_copy(acc.at[pl.ds(row0, _RSUB)], out1.at[pl.ds(row0, _RSUB)])

    return k


def kernel(node_s, node_v, edge_s, edge_v, edge_index, seq, params):
    f32 = jnp.float32
    nv3 = jnp.swapaxes(node_v, -1, -2).reshape(NN, 9).astype(f32)
    ev3 = jnp.swapaxes(edge_v, -1, -2).reshape(NE, 3).astype(f32)
    src = edge_index[0]
    dst = edge_index[1]

    tbl = _embed_node(node_s, nv3, params)
    es, ev = _embed_edge(edge_s, ev3, params)
    htok = _token_embed(seq, params['W_s'])
    idxj2, idxi2 = _dec_indices(src.reshape(1250, 128), dst.reshape(1250, 128))
    idxj = idxj2.reshape(NE)
    idxi = idxi2.reshape(NE)

    zrows = jnp.zeros((_RSUB, D_ENC), f32)
    gather_enc = _make_gather(D_ENC)
    gather_dec = _make_gather(D_DEC)
    scatter = _make_scatter()

    for li, lp in enumerate(params['enc']):
        gj, gi = gather_enc(tbl, src, dst)
        msg = _message(False, gj, gi, es, ev, lp['message'])
        a0, a1 = scatter(msg, dst, zrows)
        if li == 2:
            tbl = _node_update(D_ENC, D_DEC, True, tbl, a0, a1, htok, lp)
        else:
            tbl = _node_update(D_ENC, D_ENC, False, tbl, a0, a1, None, lp)

    encbot = jnp.concatenate([tbl[:, :C_ONE], jnp.zeros((NN, D_DEC - C_ONE), f32)],
                             axis=1)

    for li, lp in enumerate(params['dec']):
        tbl2 = jnp.concatenate([tbl, encbot], axis=0)
        gj, gi = gather_dec(tbl2, idxj, idxi)
        msg = _message(True, gj, gi, es, ev, lp['message'])
        a0, a1 = scatter(msg, dst, zrows)
        if li == 2:
            tbl = _node_update(D_DEC, D_ENC, False, tbl, a0, a1, None, lp)
        else:
            tbl = _node_update(D_DEC, D_DEC, True, tbl, a0, a1, htok, lp)

    return _readout(tbl, params)


# speedup vs baseline: 10.2005x; 1.1969x over previous
import functools
import jax
import jax.numpy as jnp
from jax import lax
from jax.experimental import pallas as pl
from jax.experimental.pallas import tpu as pltpu
from jax.experimental.pallas import tpu_sc as plsc

NN = 10000
NE = 160000
DS = 100
KV = 16
C_V = 100
C_ONE = 148
DTOK = 20
D_TBL = 256
D_MSG = 160
BN = 1000
BE = 2000
EPS = 1e-8


def _gvp_tc(v3, s_in, whT, wsT, b, wvT, relu_s, gate_v):
    vh = [x @ whT for x in v3]
    vn = jnp.sqrt(jnp.clip(vh[0] * vh[0] + vh[1] * vh[1] + vh[2] * vh[2], EPS))
    s = jnp.concatenate([s_in, vn], axis=1) @ wsT + b
    if relu_s:
        s = jnp.maximum(s, 0.0)
    vo = None
    if wvT is not None:
        vo = [h @ wvT for h in vh]
        if gate_v:
            g = jax.nn.sigmoid(jnp.sqrt(jnp.clip(
                vo[0] * vo[0] + vo[1] * vo[1] + vo[2] * vo[2], EPS)))
            vo = [x * g for x in vo]
    return s, vo


def _ln_tc(s, v3, gamma, beta):
    mu = jnp.mean(s, axis=1, keepdims=True)
    var = jnp.mean((s - mu) * (s - mu), axis=1, keepdims=True)
    s = (s - mu) * lax.rsqrt(var + 1e-5) * gamma + beta
    nsq = jnp.clip(v3[0] * v3[0] + v3[1] * v3[1] + v3[2] * v3[2], EPS)
    vn = jnp.sqrt(jnp.mean(nsq, axis=1, keepdims=True))
    return s, [x / vn for x in v3]


def _vslices(x, col, k):
    return [x[:, col + k * c:col + k * (c + 1)] for c in range(3)]


def _full(shape):
    nd = len(shape)
    return pl.BlockSpec(shape, lambda i: (0,) * nd)


def _rows(block, width):
    return pl.BlockSpec((block, width), lambda i: (i, 0))


def _gvp_w(p):
    wvT = jnp.asarray(p['wv']).T if 'wv' in p else None
    return [jnp.asarray(p['wh']).T, jnp.asarray(p['ws_w']).T,
            jnp.asarray(p['ws_b'])[None, :], wvT]


def _embed_node(node_s, nv3, params):
    w = _gvp_w(params['W_v'])
    g = params['ln_v']['gamma'][None, :]
    bt = params['ln_v']['beta'][None, :]

    def body(ns_ref, nv_ref, whT, wsT, b, wvT, lng, lnb, out_ref):
        v3 = _vslices(nv_ref[...], 0, 3)
        s, v = _gvp_tc(v3, ns_ref[...], whT[...], wsT[...], b[...], wvT[...],
                       False, False)
        s, v = _ln_tc(s, v, lng[...], lnb[...])
        out_ref[:, :DS] = s
        for c in range(3):
            out_ref[:, C_V + KV * c:C_V + KV * (c + 1)] = v[c]
        out_ref[:, C_ONE:D_TBL] = jnp.zeros((s.shape[0], D_TBL - C_ONE), jnp.float32)

    args = [node_s, nv3] + w + [g, bt]
    return pl.pallas_call(
        body, grid=(NN // BN,),
        in_specs=[_rows(BN, 6), _rows(BN, 9)] + [_full(a.shape) for a in args[2:]],
        out_specs=_rows(BN, D_TBL),
        out_shape=jax.ShapeDtypeStruct((NN, D_TBL), jnp.float32),
    )(*args)


def _embed_edge(edge_s, ev3, params):
    w = _gvp_w(params['W_e'])
    g = params['ln_e']['gamma'][None, :]
    bt = params['ln_e']['beta'][None, :]

    def body(es_ref, ev_ref, whT, wsT, b, wvT, lng, lnb, so_ref, vo_ref):
        v3 = _vslices(ev_ref[...], 0, 1)
        s, v = _gvp_tc(v3, es_ref[...], whT[...], wsT[...], b[...], wvT[...],
                       False, False)
        s, v = _ln_tc(s, v, lng[...], lnb[...])
        so_ref[...] = s
        vo_ref[...] = jnp.concatenate(v, axis=1)

    args = [edge_s, ev3] + w + [g, bt]
    return pl.pallas_call(
        body, grid=(NE // BE,),
        in_specs=[_rows(BE, 32), _rows(BE, 3)] + [_full(a.shape) for a in args[2:]],
        out_specs=(_rows(BE, 32), _rows(BE, 3)),
        out_shape=(jax.ShapeDtypeStruct((NE, 32), jnp.float32),
                   jax.ShapeDtypeStruct((NE, 3), jnp.float32)),
    )(*args)


def _token_embed(seq, W_s):
    def body(seq_ref, ws_ref, out_ref):
        oh = (lax.broadcasted_iota(jnp.int32, (BN, 33), 1) == seq_ref[...])
        out_ref[...] = oh.astype(jnp.float32) @ ws_ref[...]

    return pl.pallas_call(
        body, grid=(NN // BN,),
        in_specs=[_rows(BN, 1), _full(W_s.shape)],
        out_specs=_rows(BN, DTOK),
        out_shape=jax.ShapeDtypeStruct((NN, DTOK), jnp.float32),
    )(seq.reshape(NN, 1), W_s)


def _dec_indices(src2, dst2):
    def body(s_ref, d_ref, oj_ref, oi_ref):
        s = s_ref[...]
        d = d_ref[...]
        fut = s < d
        off = jnp.where(fut, 0, NN)
        oj_ref[...] = s + off
        oi_ref[...] = d + off

    return pl.pallas_call(
        body, grid=(1,),
        in_specs=[_rows(1250, 128), _rows(1250, 128)],
        out_specs=(_rows(1250, 128), _rows(1250, 128)),
        out_shape=(jax.ShapeDtypeStruct((1250, 128), jnp.int32),
                   jax.ShapeDtypeStruct((1250, 128), jnp.int32)),
    )(src2, dst2)


def _message(dec, gj, gi, es, ev, mp):
    w = _gvp_w(mp['g0']) + _gvp_w(mp['g1']) + _gvp_w(mp['g2'])

    def body(gj_ref, gi_ref, es_ref, ev_ref,
             wh0, ws0, b0, wv0, wh1, ws1, b1, wv1, wh2, ws2, b2, wv2,
             out_ref):
        gjv = gj_ref[...]
        giv = gi_ref[...]
        sj = gjv[:, :DS]
        si = giv[:, :DS]
        vj = _vslices(gjv, C_V, KV)
        vi = _vslices(giv, C_V, KV)
        ev_ = ev_ref[...]
        vcat = [jnp.concatenate([vj[c], ev_[:, c:c + 1], vi[c]], axis=1)
                for c in range(3)]
        if dec:
            s_in = jnp.concatenate(
                [sj, es_ref[...], gjv[:, C_ONE:C_ONE + DTOK], si], axis=1)
        else:
            s_in = jnp.concatenate([sj, es_ref[...], si], axis=1)
        s, v = _gvp_tc(vcat, s_in, wh0[...], ws0[...], b0[...], wv0[...], True, True)
        s, v = _gvp_tc(v, s, wh1[...], ws1[...], b1[...], wv1[...], True, True)
        s, v = _gvp_tc(v, s, wh2[...], ws2[...], b2[...], wv2[...], False, False)
        out_ref[:, :DS] = s
        for c in range(3):
            out_ref[:, C_V + KV * c:C_V + KV * (c + 1)] = v[c]
        B = s.shape[0]
        out_ref[:, C_ONE:C_ONE + 1] = jnp.ones((B, 1), jnp.float32)
        out_ref[:, C_ONE + 1:D_MSG] = jnp.zeros((B, D_MSG - C_ONE - 1), jnp.float32)

    args = [gj, gi, es, ev] + w
    return pl.pallas_call(
        body, grid=(NE // BE,),
        in_specs=[_rows(BE, D_TBL), _rows(BE, D_TBL), _rows(BE, 32), _rows(BE, 3)]
                 + [_full(a.shape) for a in w],
        out_specs=_rows(BE, D_MSG),
        out_shape=jax.ShapeDtypeStruct((NE, D_MSG), jnp.float32),
    )(*args)


def _node_update(Din, Dout, tok, tbl, a0, a1, htok, lp):
    w = (_gvp_w(lp['ff0']) + _gvp_w(lp['ff1'])
         + [lp['norm0']['gamma'][None, :], lp['norm0']['beta'][None, :],
            lp['norm1']['gamma'][None, :], lp['norm1']['beta'][None, :]])

    def body(*refs):
        if tok:
            (tbl_ref, a0_ref, a1_ref, tok_ref, whf0, wsf0, bf0, wvf0,
             whf1, wsf1, bf1, wvf1, g0, b0, g1, b1, out_ref) = refs
        else:
            (tbl_ref, a0_ref, a1_ref, whf0, wsf0, bf0, wvf0,
             whf1, wsf1, bf1, wvf1, g0, b0, g1, b1, out_ref) = refs
        tblv = tbl_ref[...]
        s = tblv[:, :DS]
        v = _vslices(tblv, C_V, KV)
        agg = a0_ref[...] + a1_ref[...]
        cnt = jnp.maximum(agg[:, C_ONE:C_ONE + 1], 1.0)
        s = s + agg[:, :DS] / cnt
        av = _vslices(agg, C_V, KV)
        v = [v[c] + av[c] / cnt for c in range(3)]
        s, v = _ln_tc(s, v, g0[...], b0[...])
        fs, fv = _gvp_tc(v, s, whf0[...], wsf0[...], bf0[...], wvf0[...], True, True)
        fs, fv = _gvp_tc(fv, fs, whf1[...], wsf1[...], bf1[...], wvf1[...],
                         False, False)
        s, v = _ln_tc(s + fs, [v[c] + fv[c] for c in range(3)], g1[...], b1[...])
        out_ref[:, :DS] = s
        for c in range(3):
            out_ref[:, C_V + KV * c:C_V + KV * (c + 1)] = v[c]
        if tok:
            out_ref[:, C_ONE:C_ONE + DTOK] = tok_ref[...]
            pad0 = C_ONE + DTOK
        else:
            pad0 = C_ONE
        out_ref[:, pad0:Dout] = jnp.zeros((s.shape[0], Dout - pad0), jnp.float32)

    args = [tbl, a0, a1] + ([htok] if tok else []) + w
    in_specs = ([_rows(BN, Din), _rows(BN, D_MSG), _rows(BN, D_MSG)]
                + ([_rows(BN, DTOK)] if tok else [])
                + [_full(a.shape) for a in w])
    return pl.pallas_call(
        body, grid=(NN // BN,),
        in_specs=in_specs,
        out_specs=_rows(BN, Dout),
        out_shape=jax.ShapeDtypeStruct((NN, Dout), jnp.float32),
    )(*args)


def _readout(tbl, params):
    whT = jnp.asarray(params['W_out']['wh']).T
    wsT = jnp.asarray(params['W_out']['ws_w']).T
    b = jnp.asarray(params['W_out']['ws_b'])[None, :]

    def body(tbl_ref, wh_ref, ws_ref, b_ref, lp_ref, lg_ref):
        tblv = tbl_ref[...]
        s = tblv[:, :DS]
        v = _vslices(tblv, C_V, KV)
        vh = [x @ wh_ref[...] for x in v]
        vn = jnp.sqrt(jnp.clip(vh[0] * vh[0] + vh[1] * vh[1] + vh[2] * vh[2], EPS))
        logits = jnp.concatenate([s, vn], axis=1) @ ws_ref[...] + b_ref[...]
        m = jnp.max(logits, axis=1, keepdims=True)
        lse = jnp.log(jnp.sum(jnp.exp(logits - m), axis=1, keepdims=True)) + m
        lg_ref[...] = logits
        lp_ref[...] = logits - lse

    return pl.pallas_call(
        body, grid=(NN // BN,),
        in_specs=[_rows(BN, D_TBL), _full(whT.shape), _full(wsT.shape),
                  _full(b.shape)],
        out_specs=(_rows(BN, 33), _rows(BN, 33)),
        out_shape=(jax.ShapeDtypeStruct((NN, 33), jnp.float32),
                   jax.ShapeDtypeStruct((NN, 33), jnp.float32)),
    )(tbl, whT, wsT, b)


_CH = 128
_NCHUNK = NE // _CH
_NW = 32
_PER = (_NCHUNK + _NW - 1) // _NW
_RSUB = NN // 16


def _make_gather(D):
    mesh = plsc.VectorSubcoreMesh(core_axis_name="c", subcore_axis_name="s")

    @functools.partial(
        pl.kernel,
        out_type=(jax.ShapeDtypeStruct((NE, D), jnp.float32),
                  jax.ShapeDtypeStruct((NE, D), jnp.float32)),
        mesh=mesh,
        scratch_types=[pltpu.VMEM((_CH,), jnp.int32),
                       pltpu.VMEM((_CH,), jnp.int32),
                       pltpu.VMEM((_CH, D), jnp.float32),
                       pltpu.VMEM((_CH, D), jnp.float32),
                       pltpu.SemaphoreType.DMA,
                       pltpu.SemaphoreType.DMA],
    )
    def k(tbl, idxj, idxi, gj, gi, bj, bi, rj, ri, smj, smi):
        wid = lax.axis_index("s") * 2 + lax.axis_index("c")

        def step(t, carry):
            ch = wid + t * _NW

            @pl.when(ch < _NCHUNK)
            def _():
                off = ch * _CH
                pltpu.sync_copy(idxj.at[pl.ds(off, _CH)], bj)
                pltpu.sync_copy(idxi.at[pl.ds(off, _CH)], bi)
                cj = pltpu.async_copy(tbl.at[bj], rj, smj)
                ci = pltpu.async_copy(tbl.at[bi], ri, smi)
                cj.wait()
                ci.wait()
                pltpu.sync_copy(rj, gj.at[pl.ds(off, _CH)])
                pltpu.sync_copy(ri, gi.at[pl.ds(off, _CH)])
            return carry

        lax.fori_loop(0, _PER, step, 0)

    return k


def _make_scatter():
    mesh = plsc.VectorSubcoreMesh(core_axis_name="c", subcore_axis_name="s")

    @functools.partial(
        pl.kernel,
        out_type=(jax.ShapeDtypeStruct((NN, D_MSG), jnp.float32),
                  jax.ShapeDtypeStruct((NN, D_MSG), jnp.float32)),
        mesh=mesh,
        compiler_params=pltpu.CompilerParams(use_tc_tiling_on_sc=False),
        scratch_types=[pltpu.VMEM((_CH,), jnp.int32),
                       pltpu.VMEM((_CH, D_MSG), jnp.float32),
                       pltpu.VMEM_SHARED((NN, D_MSG), jnp.float32)],
    )
    def k(msg, dst, zrows, out0, out1, bidx, bmsg, acc):
        cid = lax.axis_index("c")
        sid = lax.axis_index("s")
        wid = sid * 2 + cid
        row0 = sid * _RSUB
        pltpu.sync_copy(zrows, acc.at[pl.ds(row0, _RSUB)])
        plsc.subcore_barrier()

        def step(t, carry):
            ch = wid + t * _NW

            @pl.when(ch < _NCHUNK)
            def _():
                off = ch * _CH
                pltpu.sync_copy(dst.at[pl.ds(off, _CH)], bidx)
                pltpu.sync_copy(msg.at[pl.ds(off, _CH)], bmsg)
                pltpu.sync_copy(bmsg, acc.at[bidx], add=True)
            return carry

        lax.fori_loop(0, _PER, step, 0)
        plsc.subcore_barrier()

        @pl.when(cid == 0)
        def _():
            pltpu.sync_copy(acc.at[pl.ds(row0, _RSUB)], out0.at[pl.ds(row0, _RSUB)])

        @pl.when(cid == 1)
        def _():
            pltpu.sync_copy(acc.at[pl.ds(row0, _RSUB)], out1.at[pl.ds(row0, _RSUB)])

    return k


def kernel(node_s, node_v, edge_s, edge_v, edge_index, seq, params):
    f32 = jnp.float32
    nv3 = jnp.swapaxes(node_v, -1, -2).reshape(NN, 9).astype(f32)
    ev3 = jnp.swapaxes(edge_v, -1, -2).reshape(NE, 3).astype(f32)
    src = edge_index[0]
    dst = edge_index[1]

    tbl = _embed_node(node_s, nv3, params)
    es, ev = _embed_edge(edge_s, ev3, params)
    htok = _token_embed(seq, params['W_s'])
    idxj2, idxi2 = _dec_indices(src.reshape(1250, 128), dst.reshape(1250, 128))
    idxj = idxj2.reshape(NE)
    idxi = idxi2.reshape(NE)

    zrows = jnp.zeros((_RSUB, D_MSG), f32)
    gather = _make_gather(D_TBL)
    scatter = _make_scatter()

    for li, lp in enumerate(params['enc']):
        gj, gi = gather(tbl, src, dst)
        msg = _message(False, gj, gi, es, ev, lp['message'])
        a0, a1 = scatter(msg, dst, zrows)
        if li == 2:
            tbl = _node_update(D_TBL, D_TBL, True, tbl, a0, a1, htok, lp)
        else:
            tbl = _node_update(D_TBL, D_TBL, False, tbl, a0, a1, None, lp)

    encbot = jnp.concatenate([tbl[:, :C_ONE], jnp.zeros((NN, D_TBL - C_ONE), f32)],
                             axis=1)

    for li, lp in enumerate(params['dec']):
        tbl2 = jnp.concatenate([tbl, encbot], axis=0)
        gj, gi = gather(tbl2, idxj, idxi)
        msg = _message(True, gj, gi, es, ev, lp['message'])
        a0, a1 = scatter(msg, dst, zrows)
        if li == 2:
            tbl = _node_update(D_TBL, D_TBL, False, tbl, a0, a1, None, lp)
        else:
            tbl = _node_update(D_TBL, D_TBL, True, tbl, a0, a1, htok, lp)

    return _readout(tbl, params)
